# Initial kernel scaffold; baseline (speedup 1.0000x reference)
import jax
import jax.numpy as jnp
from jax.experimental import pallas as pl


def kernel(token_ids, weight):
    raise NotImplementedError("write your pallas kernel here")


# speedup vs baseline: 1.4779x; 1.4779x over previous
import functools

import jax
import jax.numpy as jnp
from jax import lax
from jax.experimental import pallas as pl
from jax.experimental.pallas import tpu as pltpu
from jax.experimental.pallas import tpu_sc as plsc

NC = 2
NS = 16
NW = NC * NS

B = 4096 * 200
D = 32
BPW = B // NW
CHUNK = 1600
NCHUNK = BPW // CHUNK

_MESH = plsc.VectorSubcoreMesh(
    core_axis_name="c", subcore_axis_name="s", num_cores=NC, num_subcores=NS
)


@functools.partial(
    pl.kernel,
    out_type=jax.ShapeDtypeStruct((B, D), jnp.float32),
    mesh=_MESH,
    scratch_types=[
        pltpu.VMEM((CHUNK,), jnp.int32),
        pltpu.VMEM((CHUNK, D), jnp.float32),
        pltpu.SemaphoreType.DMA,
    ],
    compiler_params=pltpu.CompilerParams(use_tc_tiling_on_sc=False),
)
def _embed_gather(idx_hbm, table_hbm, out_hbm, idx_v, rows_v, sem):
    wid = lax.axis_index("s") * NC + lax.axis_index("c")
    base = wid * BPW

    def body(j, carry):
        off = base + j * CHUNK
        pltpu.sync_copy(idx_hbm.at[pl.ds(off, CHUNK)], idx_v)
        pltpu.async_copy(table_hbm.at[idx_v], rows_v, sem).wait()
        pltpu.sync_copy(rows_v, out_hbm.at[pl.ds(off, CHUNK)])
        return carry

    lax.fori_loop(0, NCHUNK, body, 0)


def kernel(token_ids, weight):
    flat = token_ids.reshape(-1).astype(jnp.int32)
    out = _embed_gather(flat, weight)
    return out.reshape(token_ids.shape + (weight.shape[1],))


# speedup vs baseline: 1.4915x; 1.0092x over previous
import functools

import jax
import jax.numpy as jnp
from jax import lax
from jax.experimental import pallas as pl
from jax.experimental.pallas import tpu as pltpu
from jax.experimental.pallas import tpu_sc as plsc

NC = 2
NS = 16
NW = NC * NS

B = 4096 * 200
D = 32
BPW = B // NW
CHUNK = 1600
NCHUNK = BPW // CHUNK

_MESH = plsc.VectorSubcoreMesh(
    core_axis_name="c", subcore_axis_name="s", num_cores=NC, num_subcores=NS
)


@functools.partial(
    pl.kernel,
    out_type=jax.ShapeDtypeStruct((B, D), jnp.float32),
    mesh=_MESH,
    scratch_types=[
        pltpu.VMEM((CHUNK,), jnp.int32),
        pltpu.VMEM((CHUNK,), jnp.int32),
        pltpu.VMEM((CHUNK, D), jnp.float32),
        pltpu.VMEM((CHUNK, D), jnp.float32),
        pltpu.SemaphoreType.DMA,
        pltpu.SemaphoreType.DMA,
        pltpu.SemaphoreType.DMA,
        pltpu.SemaphoreType.DMA,
        pltpu.SemaphoreType.DMA,
        pltpu.SemaphoreType.DMA,
    ],
    compiler_params=pltpu.CompilerParams(use_tc_tiling_on_sc=False),
)
def _embed_gather(idx_hbm, table_hbm, out_hbm, idx0, idx1, rows0, rows1,
                  isem0, isem1, gsem0, gsem1, wsem0, wsem1):
    wid = lax.axis_index("s") * NC + lax.axis_index("c")
    base = wid * BPW

    idxb = (idx0, idx1)
    rowsb = (rows0, rows1)
    isems = (isem0, isem1)
    gsems = (gsem0, gsem1)
    wsems = (wsem0, wsem1)

    def idx_copy(j):
        b = j & 1
        src = idx_hbm.at[pl.ds(base + j * CHUNK, CHUNK)]
        return pltpu.make_async_copy(src, idxb[b], isems[b])

    def gather(j):
        b = j & 1
        return pltpu.make_async_copy(table_hbm.at[idxb[b]], rowsb[b], gsems[b])

    def writeout(j):
        b = j & 1
        dst = out_hbm.at[pl.ds(base + j * CHUNK, CHUNK)]
        return pltpu.make_async_copy(rowsb[b], dst, wsems[b])

    idx_copy(0).start()
    for j in range(NCHUNK):
        if j > 0:
            gather(j - 1).wait()
            writeout(j - 1).start()
        if j + 1 < NCHUNK:
            idx_copy(j + 1).start()
        idx_copy(j).wait()
        if j >= 2:
            writeout(j - 2).wait()
        gather(j).start()
    gather(NCHUNK - 1).wait()
    writeout(NCHUNK - 1).start()
    writeout(NCHUNK - 2).wait()
    writeout(NCHUNK - 1).wait()


def kernel(token_ids, weight):
    flat = token_ids.reshape(-1).astype(jnp.int32)
    out = _embed_gather(flat, weight)
    return out.reshape(token_ids.shape + (weight.shape[1],))


# speedup vs baseline: 1.4993x; 1.0052x over previous
import functools

import jax
import jax.numpy as jnp
from jax import lax
from jax.experimental import pallas as pl
from jax.experimental.pallas import tpu as pltpu
from jax.experimental.pallas import tpu_sc as plsc

NC = 2
NS = 16
NW = NC * NS

B = 4096 * 200
D = 32
BPW = B // NW
NBUF = 4
CHUNK = 800
NCHUNK = BPW // CHUNK

_MESH = plsc.VectorSubcoreMesh(
    core_axis_name="c", subcore_axis_name="s", num_cores=NC, num_subcores=NS
)

_SCRATCH = (
    [pltpu.VMEM((CHUNK,), jnp.int32) for _ in range(NBUF)]
    + [pltpu.VMEM((CHUNK, D), jnp.float32) for _ in range(NBUF)]
    + [pltpu.SemaphoreType.DMA for _ in range(3 * NBUF)]
)


@functools.partial(
    pl.kernel,
    out_type=jax.ShapeDtypeStruct((B, D), jnp.float32),
    mesh=_MESH,
    scratch_types=_SCRATCH,
    compiler_params=pltpu.CompilerParams(use_tc_tiling_on_sc=False),
)
def _embed_gather(idx_hbm, table_hbm, out_hbm, *scratch):
    idxb = scratch[:NBUF]
    rowsb = scratch[NBUF:2 * NBUF]
    isems = scratch[2 * NBUF:3 * NBUF]
    gsems = scratch[3 * NBUF:4 * NBUF]
    wsems = scratch[4 * NBUF:5 * NBUF]

    wid = lax.axis_index("s") * NC + lax.axis_index("c")
    base = wid * BPW

    def idx_copy(j):
        b = j % NBUF
        src = idx_hbm.at[pl.ds(base + j * CHUNK, CHUNK)]
        return pltpu.make_async_copy(src, idxb[b], isems[b])

    def gather(j):
        b = j % NBUF
        return pltpu.make_async_copy(table_hbm.at[idxb[b]], rowsb[b], gsems[b])

    def writeout(j):
        b = j % NBUF
        dst = out_hbm.at[pl.ds(base + j * CHUNK, CHUNK)]
        return pltpu.make_async_copy(rowsb[b], dst, wsems[b])

    for j in range(min(NBUF, NCHUNK)):
        idx_copy(j).start()
    for j in range(NCHUNK):
        idx_copy(j).wait()
        if j >= NBUF:
            writeout(j - NBUF).wait()
        gather(j).start()
        k = j - (NBUF - 1)
        if k >= 0:
            gather(k).wait()
            writeout(k).start()
            if k + NBUF < NCHUNK:
                idx_copy(k + NBUF).start()
    for k in range(max(0, NCHUNK - NBUF + 1), NCHUNK):
        gather(k).wait()
        writeout(k).start()
    for k in range(max(0, NCHUNK - NBUF), NCHUNK):
        writeout(k).wait()


def kernel(token_ids, weight):
    flat = token_ids.reshape(-1).astype(jnp.int32)
    out = _embed_gather(flat, weight)
    return out.reshape(token_ids.shape + (weight.shape[1],))
